# baseline (speedup 1.0000x reference)
import functools

import jax
import jax.numpy as jnp
from jax import lax
from jax.experimental import pallas as pl
from jax.experimental.pallas import tpu as pltpu
from jax.experimental.pallas import tpu_sc as plsc

NC = 2
NS = 16
NW = NC * NS
L = 16

B = 16384
D = 32
ROWS_PER_TILE = 128 // D
BPW = B // NW
CHUNK = 128
NCHUNK = BPW // CHUNK
GPC = CHUNK // L


def _body(uid_hbm, iid_hbm, ut_hbm, it_hbm, out_hbm,
          idx_u, idx_i, hi_u, hi_i, u_bufs, i_bufs, out_v, sems):
    wid = lax.axis_index("s") * NC + lax.axis_index("c")
    base = wid * BPW

    for c in range(NCHUNK):
        pltpu.sync_copy(uid_hbm.at[wid * NCHUNK + c], idx_u.at[c])
        pltpu.sync_copy(iid_hbm.at[wid * NCHUNK + c], idx_i.at[c])

    def split(c):
        for k in range(GPC):
            sl = pl.ds(k * L, L)
            hi_u[c, sl] = idx_u[c, sl] >> 2
            hi_i[c, sl] = idx_i[c, sl] >> 2

    def fire(c):
        buf = c % 2
        cu = pltpu.async_copy(ut_hbm.at[hi_u.at[c]], u_bufs.at[buf],
                              sems.at[buf])
        ci = pltpu.async_copy(it_hbm.at[hi_i.at[c]], i_bufs.at[buf],
                              sems.at[2 + buf])
        return cu, ci

    split(0)
    inflight = {0: fire(0)}
    split(1)
    inflight[1] = fire(1)
    split(2)
    split(3)

    iota = lax.iota(jnp.int32, L)

    for c in range(NCHUNK):
        buf = c % 2
        cu, ci = inflight.pop(c)
        cu.wait()
        ci.wait()

        def group(g, carry, c=c, buf=buf):
            sl = pl.ds(g * L, L)
            row = g * L + iota
            lo_u = (idx_u[c, sl] & (ROWS_PER_TILE - 1)) * D
            lo_i = (idx_i[c, sl] & (ROWS_PER_TILE - 1)) * D
            acc = jnp.zeros((L,), jnp.float32)
            for d in range(D):
                u = plsc.load_gather(u_bufs.at[buf], [row, lo_u + d])
                v = plsc.load_gather(i_bufs.at[buf], [row, lo_i + d])
                acc = acc + u * v
            out_v[pl.ds(c * CHUNK + g * L, L)] = acc
            return carry

        lax.fori_loop(0, GPC, group, None)
        if c + 2 < NCHUNK:
            inflight[c + 2] = fire(c + 2)

    pltpu.sync_copy(out_v, out_hbm.at[pl.ds(base, BPW)])


@functools.cache
def _build():
    return pl.kernel(
        _body,
        out_type=jax.ShapeDtypeStruct((B,), jnp.float32),
        mesh=plsc.VectorSubcoreMesh(core_axis_name="c", subcore_axis_name="s",
                                    num_cores=NC, num_subcores=NS),
        compiler_params=pltpu.CompilerParams(needs_layout_passes=False),
        scratch_types=[
            pltpu.VMEM((NCHUNK, CHUNK), jnp.int32),
            pltpu.VMEM((NCHUNK, CHUNK), jnp.int32),
            pltpu.VMEM((NCHUNK, CHUNK), jnp.int32),
            pltpu.VMEM((NCHUNK, CHUNK), jnp.int32),
            pltpu.VMEM((2, CHUNK, 128), jnp.float32),
            pltpu.VMEM((2, CHUNK, 128), jnp.float32),
            pltpu.VMEM((BPW,), jnp.float32),
            pltpu.SemaphoreType.DMA((4,)),
        ],
    )


@jax.jit
def kernel(user_id, item_id, user_table, item_table):
    uid = user_id.astype(jnp.int32).reshape(NW * NCHUNK, CHUNK)
    iid = item_id.astype(jnp.int32).reshape(NW * NCHUNK, CHUNK)
    ut = user_table.reshape(-1, 128)
    it = item_table.reshape(-1, 128)
    return _build()(uid, iid, ut, it)
